# baseline (speedup 1.0000x reference)
import jax
import jax.numpy as jnp
from jax.experimental import pallas as pl
from jax.experimental.pallas import tpu as pltpu

_NUM_OUT = 1000
_BLOCK_B = 1024
_WCHUNK = 1024


def _make_kernel(ncast, bm):
    def _fused_mlp_kernel(x_ref, w1c_ref, w2c_ref, b2_ref, ot_ref,
                          w1b_ref, w2b_ref):
        i = pl.program_id(0)

        @pl.when(i < ncast)
        def _cast_phase():
            lo = pl.multiple_of(i * _WCHUNK, _WCHUNK)
            w1b_ref[:, pl.ds(lo, _WCHUNK)] = w1c_ref[...].astype(jnp.bfloat16)
            w2b_ref[pl.ds(lo, _WCHUNK), :] = w2c_ref[...].astype(jnp.bfloat16)

        @pl.when(i >= ncast)
        def _compute_phase():
            x = x_ref[...].astype(jnp.bfloat16)
            h = jnp.dot(x, w1b_ref[...], preferred_element_type=jnp.float32)
            h = jnp.maximum(h, 0.0).astype(jnp.bfloat16)
            out = jnp.dot(h, w2b_ref[...], preferred_element_type=jnp.float32)
            out = out + b2_ref[...]
            ot_ref[...] = out.T[:_NUM_OUT].astype(ot_ref.dtype)

    return _fused_mlp_kernel


@jax.jit
def kernel(x, w1_p, w2_p, b2_p):
    B, Din = x.shape
    Hp = w1_p.shape[1]
    Cp = w2_p.shape[1]

    bm = _BLOCK_B if B % _BLOCK_B == 0 else B
    ncast = Hp // _WCHUNK
    ncomp = B // bm

    out_t = pl.pallas_call(
        _make_kernel(ncast, bm),
        out_shape=jax.ShapeDtypeStruct((_NUM_OUT, B), x.dtype),
        grid=(ncast + ncomp,),
        in_specs=[
            pl.BlockSpec((bm, Din), lambda i: (jnp.maximum(i - ncast, 0), 0)),
            pl.BlockSpec((Din, _WCHUNK), lambda i: (0, jnp.minimum(i, ncast - 1))),
            pl.BlockSpec((_WCHUNK, Cp), lambda i: (jnp.minimum(i, ncast - 1), 0)),
            pl.BlockSpec((1, Cp), lambda i: (0, 0)),
        ],
        out_specs=pl.BlockSpec((_NUM_OUT, bm),
                               lambda i: (0, jnp.maximum(i - ncast, 0))),
        scratch_shapes=[
            pltpu.VMEM((Din, Hp), jnp.bfloat16),
            pltpu.VMEM((Hp, Cp), jnp.bfloat16),
        ],
        compiler_params=pltpu.CompilerParams(
            dimension_semantics=("arbitrary",)),
    )(x, w1_p, w2_p, b2_p)
    return out_t.T


# speedup vs baseline: 1.0092x; 1.0092x over previous
import jax
import jax.numpy as jnp
from jax.experimental import pallas as pl
from jax.experimental.pallas import tpu as pltpu

_NUM_OUT = 1000
_BLOCK_B = 512


def _fused_mlp_kernel(x_ref, w1_ref, w2_ref, b2_ref, ot_ref):
    h = jnp.dot(x_ref[...], w1_ref[...], preferred_element_type=jnp.float32)
    h = jnp.maximum(h, 0.0)
    out = jnp.dot(h, w2_ref[...], preferred_element_type=jnp.float32)
    out = out + b2_ref[...]
    ot_ref[...] = out.T[:_NUM_OUT].astype(ot_ref.dtype)


@jax.jit
def kernel(x, w1_p, w2_p, b2_p):
    B, Din = x.shape
    Hp = w1_p.shape[1]
    Cp = w2_p.shape[1]
    bm = _BLOCK_B if B % _BLOCK_B == 0 else B
    out_t = pl.pallas_call(
        _fused_mlp_kernel,
        out_shape=jax.ShapeDtypeStruct((_NUM_OUT, B), x.dtype),
        grid=(B // bm,),
        in_specs=[
            pl.BlockSpec((bm, Din), lambda i: (i, 0)),
            pl.BlockSpec((Din, Hp), lambda i: (0, 0)),
            pl.BlockSpec((Hp, Cp), lambda i: (0, 0)),
            pl.BlockSpec((1, Cp), lambda i: (0, 0)),
        ],
        out_specs=pl.BlockSpec((_NUM_OUT, bm), lambda i: (0, i)),
        compiler_params=pltpu.CompilerParams(
            dimension_semantics=("arbitrary",)),
    )(x, w1_p, w2_p, b2_p)
    return out_t.T
